# Initial kernel scaffold; baseline (speedup 1.0000x reference)
import jax
import jax.numpy as jnp
from jax.experimental import pallas as pl


def kernel(x, edge_index, W1l, W1r, a1, b1, W2l, W2r, a2, b2):
    raise NotImplementedError("write your pallas kernel here")


# speedup vs baseline: 5.0463x; 5.0463x over previous
import functools

import jax
import jax.numpy as jnp
from jax import lax
from jax.experimental import pallas as pl
from jax.experimental.pallas import tpu as pltpu
from jax.experimental.pallas import tpu_sc as plsc

N = 10000
E = 320000
D_IN = 128
D = 32

NC = 2
NS = 16
NW = NC * NS
NP = 10240
EPW = E // NW
C = 80
NCHUNK = EPW // C
G = C // 16
RPS = NP // NS

_f32 = jnp.float32
_i32 = jnp.int32

_MESH = plsc.VectorSubcoreMesh(core_axis_name="c", subcore_axis_name="s")
_SC_PARAMS = pltpu.CompilerParams(
    needs_layout_passes=False, use_tc_tiling_on_sc=False)


def _edge_alpha_body(src_hbm, dst_hbm, xl_hbm, xr_hbm, att_hbm, zeros_hbm,
                     alpha_hbm, seg_hbm,
                     seg_sh, att_v, idx_s, idx_d, rows_s, rows_d, alpha_v,
                     sem):
    c = lax.axis_index("c")
    s = lax.axis_index("s")
    wid = s * NC + c
    ebase = wid * EPW

    pltpu.sync_copy(zeros_hbm.at[pl.ds(0, RPS)], seg_sh.at[pl.ds(s * RPS, RPS)])
    pltpu.sync_copy(att_hbm, att_v)
    plsc.subcore_barrier()

    a_lo = att_v[pl.ds(0, 16)]
    a_hi = att_v[pl.ds(16, 16)]
    atts = [a_lo[f] for f in range(16)] + [a_hi[f] for f in range(16)]

    def body(i, carry):
        base = ebase + i * C
        pltpu.sync_copy(src_hbm.at[pl.ds(base, C)], idx_s)
        pltpu.sync_copy(dst_hbm.at[pl.ds(base, C)], idx_d)
        cp1 = pltpu.async_copy(xl_hbm.at[idx_s], rows_s, sem)
        cp2 = pltpu.async_copy(xr_hbm.at[idx_d], rows_d, sem)
        cp1.wait()
        cp2.wait()
        for g in range(G):
            eids = lax.iota(_i32, 16) + _i32(g * 16)
            acc = jnp.zeros((16,), _f32)
            for f in range(D):
                fv = jnp.full((16,), f, _i32)
                sv = (plsc.load_gather(rows_s, [eids, fv])
                      + plsc.load_gather(rows_d, [eids, fv]))
                m = jnp.maximum(sv, 0.2 * sv)
                acc = acc + atts[f] * m
            alpha_v[pl.ds(g * 16, 16)] = jnp.exp(acc)
        pltpu.sync_copy(alpha_v, alpha_hbm.at[pl.ds(base, C)])
        pltpu.sync_copy(alpha_v, seg_sh.at[idx_d], add=True)
        return carry

    lax.fori_loop(0, NCHUNK, body, 0)

    plsc.subcore_barrier()
    pltpu.sync_copy(seg_sh.at[pl.ds(s * RPS, RPS)],
                    seg_hbm.at[pl.ds(c * NP + s * RPS, RPS)])


_edge_alpha = functools.partial(
    pl.kernel,
    out_type=[jax.ShapeDtypeStruct((E,), _f32),
              jax.ShapeDtypeStruct((NC * NP,), _f32)],
    mesh=_MESH,
    scratch_types=[
        pltpu.VMEM_SHARED((NP,), _f32),
        pltpu.VMEM((D,), _f32),
        pltpu.VMEM((C,), _i32),
        pltpu.VMEM((C,), _i32),
        pltpu.VMEM((C, D), _f32),
        pltpu.VMEM((C, D), _f32),
        pltpu.VMEM((C,), _f32),
        pltpu.SemaphoreType.DMA,
    ],
    compiler_params=_SC_PARAMS,
)(_edge_alpha_body)


def _edge_aggregate_body(src_hbm, dst_hbm, alpha_hbm, seg_hbm, xl_hbm,
                         zeros_hbm,
                         outp_hbm,
                         out_sh, rsum_v, segb_v, idx_s, idx_d, rows_v,
                         wrows_v, alpha_v, sem):
    c = lax.axis_index("c")
    s = lax.axis_index("s")
    wid = s * NC + c
    ebase = wid * EPW

    pltpu.sync_copy(zeros_hbm, out_sh.at[pl.ds(s * RPS, RPS)])

    pltpu.sync_copy(seg_hbm.at[pl.ds(0, NP)], rsum_v)
    pltpu.sync_copy(seg_hbm.at[pl.ds(NP, NP)], segb_v)

    def rbody(i, carry):
        a = rsum_v[pl.ds(i * 16, 16)]
        b = segb_v[pl.ds(i * 16, 16)]
        rsum_v[pl.ds(i * 16, 16)] = 1.0 / (a + b + 1e-16)
        return carry

    lax.fori_loop(0, NP // 16, rbody, 0)
    plsc.subcore_barrier()

    def body(i, carry):
        base = ebase + i * C
        pltpu.sync_copy(src_hbm.at[pl.ds(base, C)], idx_s)
        pltpu.sync_copy(dst_hbm.at[pl.ds(base, C)], idx_d)
        pltpu.sync_copy(alpha_hbm.at[pl.ds(base, C)], alpha_v)
        pltpu.async_copy(xl_hbm.at[idx_s], rows_v, sem).wait()
        for g in range(G):
            eids = lax.iota(_i32, 16) + _i32(g * 16)
            dv = idx_d[pl.ds(g * 16, 16)]
            w = alpha_v[pl.ds(g * 16, 16)] * plsc.load_gather(rsum_v, [dv])
            for f in range(D):
                fv = jnp.full((16,), f, _i32)
                v = plsc.load_gather(rows_v, [eids, fv]) * w
                plsc.store_scatter(wrows_v, [eids, fv], v)
        pltpu.sync_copy(wrows_v, out_sh.at[idx_d], add=True)
        return carry

    lax.fori_loop(0, NCHUNK, body, 0)

    plsc.subcore_barrier()
    pltpu.sync_copy(out_sh.at[pl.ds(s * RPS, RPS)],
                    outp_hbm.at[pl.ds(c * NP + s * RPS, RPS)])


_edge_aggregate = functools.partial(
    pl.kernel,
    out_type=jax.ShapeDtypeStruct((NC * NP, D), _f32),
    mesh=_MESH,
    scratch_types=[
        pltpu.VMEM_SHARED((NP, D), _f32),
        pltpu.VMEM((NP,), _f32),
        pltpu.VMEM((NP,), _f32),
        pltpu.VMEM((C,), _i32),
        pltpu.VMEM((C,), _i32),
        pltpu.VMEM((C, D), _f32),
        pltpu.VMEM((C, D), _f32),
        pltpu.VMEM((C,), _f32),
        pltpu.SemaphoreType.DMA,
    ],
    compiler_params=_SC_PARAMS,
)(_edge_aggregate_body)


def _mm2_body(x_ref, wl_ref, wr_ref, xl_ref, xr_ref):
    x = x_ref[...]
    xl_ref[...] = jnp.dot(x, wl_ref[...], preferred_element_type=_f32)
    xr_ref[...] = jnp.dot(x, wr_ref[...], preferred_element_type=_f32)


def _layer2_body(p_ref, b_ref, wl_ref, wr_ref, xl_ref, xr_ref):
    p = p_ref[...]
    h = jnp.maximum(p[:NP] + p[NP:] + b_ref[...], 0.0)
    xl_ref[...] = jnp.dot(h, wl_ref[...], preferred_element_type=_f32)
    xr_ref[...] = jnp.dot(h, wr_ref[...], preferred_element_type=_f32)


def _final_body(p_ref, b_ref, o_ref):
    p = p_ref[...]
    o_ref[...] = p[:NP] + p[NP:] + b_ref[...]


def kernel(x, edge_index, W1l, W1r, a1, b1, W2l, W2r, a2, b2):
    src = edge_index[0].astype(_i32)
    dst = edge_index[1].astype(_i32)
    xpad = jnp.pad(x.astype(_f32), ((0, NP - N), (0, 0)))
    zeros_rows = jnp.zeros((RPS, D), _f32)
    zeros_flat = jnp.zeros((RPS,), _f32)

    xl1, xr1 = pl.pallas_call(
        _mm2_body,
        out_shape=[jax.ShapeDtypeStruct((NP, D), _f32)] * 2,
    )(xpad, W1l, W1r)

    alpha1, seg1 = _edge_alpha(src, dst, xl1, xr1, a1, zeros_flat)
    part1 = _edge_aggregate(src, dst, alpha1, seg1, xl1, zeros_rows)

    xl2, xr2 = pl.pallas_call(
        _layer2_body,
        out_shape=[jax.ShapeDtypeStruct((NP, D), _f32)] * 2,
    )(part1, b1.reshape(1, D), W2l, W2r)

    alpha2, seg2 = _edge_alpha(src, dst, xl2, xr2, a2, zeros_flat)
    part2 = _edge_aggregate(src, dst, alpha2, seg2, xl2, zeros_rows)

    out = pl.pallas_call(
        _final_body,
        out_shape=jax.ShapeDtypeStruct((NP, D), _f32),
    )(part2, b2.reshape(1, D))
    return out[:N]


# speedup vs baseline: 7.9444x; 1.5743x over previous
import functools

import jax
import jax.numpy as jnp
from jax import lax
from jax.experimental import pallas as pl
from jax.experimental.pallas import tpu as pltpu
from jax.experimental.pallas import tpu_sc as plsc

N = 10000
E = 320000
D_IN = 128
D = 32

NC = 2
NS = 16
NW = NC * NS
NP = 10240
EPW = E // NW
C = 80
NCHUNK = EPW // C
NDBL = (NCHUNK - 1) // 2
G = C // 16
RPS = NP // NS

_f32 = jnp.float32
_i32 = jnp.int32

_MESH = plsc.VectorSubcoreMesh(core_axis_name="c", subcore_axis_name="s")
_SC_PARAMS = pltpu.CompilerParams(
    needs_layout_passes=False, use_tc_tiling_on_sc=False)


def _edge_body(src_hbm, dst_hbm, xl_hbm, xr_hbm, att_hbm, zrows_hbm,
               zflat_hbm,
               num_hbm, seg_hbm,
               num_sh, seg_sh, att_v, idxs_all, idxd_all,
               isc0, idc0, isc1, idc1,
               rows_s0, rows_d0, rows_s1, rows_d1,
               wrows0, wrows1, alc0, alc1,
               sem0, sem1):
    c = lax.axis_index("c")
    s = lax.axis_index("s")
    ebase = (s * NC + c) * EPW

    pltpu.sync_copy(zrows_hbm, num_sh.at[pl.ds(s * RPS, RPS)])
    pltpu.sync_copy(zflat_hbm, seg_sh.at[pl.ds(s * RPS, RPS)])
    pltpu.sync_copy(att_hbm, att_v)
    pltpu.sync_copy(src_hbm.at[pl.ds(ebase, EPW)], idxs_all)
    pltpu.sync_copy(dst_hbm.at[pl.ds(ebase, EPW)], idxd_all)
    plsc.subcore_barrier()

    a_lo = att_v[pl.ds(0, 16)]
    a_hi = att_v[pl.ds(16, 16)]
    atts = [a_lo[f] for f in range(16)] + [a_hi[f] for f in range(16)]

    bufs = [(isc0, idc0, rows_s0, rows_d0, wrows0, alc0, sem0),
            (isc1, idc1, rows_s1, rows_d1, wrows1, alc1, sem1)]

    def issue(i, p):
        isc, idc, rs, rd, _, _, sm = bufs[p]
        for g in range(G):
            isc[pl.ds(g * 16, 16)] = idxs_all[pl.ds(i * C + g * 16, 16)]
            idc[pl.ds(g * 16, 16)] = idxd_all[pl.ds(i * C + g * 16, 16)]
        pltpu.async_copy(xl_hbm.at[isc], rs, sm)
        pltpu.async_copy(xr_hbm.at[idc], rd, sm)

    def wait(p):
        isc, idc, rs, rd, _, _, sm = bufs[p]
        pltpu.make_async_copy(xl_hbm.at[isc], rs, sm).wait()
        pltpu.make_async_copy(xr_hbm.at[idc], rd, sm).wait()

    def process(i, p):
        isc, idc, rs, rd, wrows, alc, sm = bufs[p]
        for g in range(G):
            eids = lax.iota(_i32, 16) + _i32(g * 16)
            accs = [jnp.zeros((16,), _f32) for _ in range(4)]
            for f in range(D):
                fv = jnp.full((16,), f, _i32)
                sv = (plsc.load_gather(rs, [eids, fv])
                      + plsc.load_gather(rd, [eids, fv]))
                m = jnp.maximum(sv, 0.2 * sv)
                accs[f % 4] = accs[f % 4] + atts[f] * m
            alpha = jnp.exp((accs[0] + accs[1]) + (accs[2] + accs[3]))
            alc[pl.ds(g * 16, 16)] = alpha
            for f in range(D):
                fv = jnp.full((16,), f, _i32)
                v = plsc.load_gather(rs, [eids, fv]) * alpha
                plsc.store_scatter(wrows, [eids, fv], v)
        pltpu.sync_copy(alc, seg_sh.at[idc], add=True)
        pltpu.sync_copy(wrows, num_sh.at[idc], add=True)

    issue(0, 0)

    def body(j, carry):
        i0 = 2 * j
        wait(0)
        issue(i0 + 1, 1)
        process(i0, 0)
        wait(1)
        issue(i0 + 2, 0)
        process(i0 + 1, 1)
        return carry

    lax.fori_loop(0, NDBL, body, 0)
    wait(0)
    process(NCHUNK - 1, 0)

    plsc.subcore_barrier()
    pltpu.sync_copy(num_sh.at[pl.ds(s * RPS, RPS)],
                    num_hbm.at[pl.ds(c * NP + s * RPS, RPS)])
    pltpu.sync_copy(seg_sh.at[pl.ds(s * RPS, RPS)],
                    seg_hbm.at[pl.ds(c * NP + s * RPS, RPS)])


_edge_layer = functools.partial(
    pl.kernel,
    out_type=[jax.ShapeDtypeStruct((NC * NP, D), _f32),
              jax.ShapeDtypeStruct((NC * NP,), _f32)],
    mesh=_MESH,
    scratch_types=[
        pltpu.VMEM_SHARED((NP, D), _f32),
        pltpu.VMEM_SHARED((NP,), _f32),
        pltpu.VMEM((D,), _f32),
        pltpu.VMEM((EPW,), _i32),
        pltpu.VMEM((EPW,), _i32),
        pltpu.VMEM((C,), _i32),
        pltpu.VMEM((C,), _i32),
        pltpu.VMEM((C,), _i32),
        pltpu.VMEM((C,), _i32),
        pltpu.VMEM((C, D), _f32),
        pltpu.VMEM((C, D), _f32),
        pltpu.VMEM((C, D), _f32),
        pltpu.VMEM((C, D), _f32),
        pltpu.VMEM((C, D), _f32),
        pltpu.VMEM((C, D), _f32),
        pltpu.VMEM((C,), _f32),
        pltpu.VMEM((C,), _f32),
        pltpu.SemaphoreType.DMA,
        pltpu.SemaphoreType.DMA,
    ],
    compiler_params=_SC_PARAMS,
)(_edge_body)


def _mm2_body(x_ref, wl_ref, wr_ref, xl_ref, xr_ref):
    x = x_ref[...]
    xl_ref[...] = jnp.dot(x, wl_ref[...], preferred_element_type=_f32)
    xr_ref[...] = jnp.dot(x, wr_ref[...], preferred_element_type=_f32)


def _layer2_body(p_ref, s0_ref, s1_ref, b_ref, wl_ref, wr_ref,
                 xl_ref, xr_ref):
    p = p_ref[...]
    seg = s0_ref[...] + s1_ref[...] + 1e-16
    h = jnp.maximum((p[:NP] + p[NP:]) / seg + b_ref[...], 0.0)
    xl_ref[...] = jnp.dot(h, wl_ref[...], preferred_element_type=_f32)
    xr_ref[...] = jnp.dot(h, wr_ref[...], preferred_element_type=_f32)


def _final_body(p_ref, s0_ref, s1_ref, b_ref, o_ref):
    p = p_ref[...]
    seg = s0_ref[...] + s1_ref[...] + 1e-16
    o_ref[...] = (p[:NP] + p[NP:]) / seg + b_ref[...]


def kernel(x, edge_index, W1l, W1r, a1, b1, W2l, W2r, a2, b2):
    src = edge_index[0].astype(_i32)
    dst = edge_index[1].astype(_i32)
    xpad = jnp.pad(x.astype(_f32), ((0, NP - N), (0, 0)))
    zeros_rows = jnp.zeros((RPS, D), _f32)
    zeros_flat = jnp.zeros((RPS,), _f32)

    xl1, xr1 = pl.pallas_call(
        _mm2_body,
        out_shape=[jax.ShapeDtypeStruct((NP, D), _f32)] * 2,
    )(xpad, W1l, W1r)

    num1, seg1 = _edge_layer(src, dst, xl1, xr1, a1, zeros_rows, zeros_flat)
    s1a = seg1[:NP, None]
    s1b = seg1[NP:, None]

    xl2, xr2 = pl.pallas_call(
        _layer2_body,
        out_shape=[jax.ShapeDtypeStruct((NP, D), _f32)] * 2,
    )(num1, s1a, s1b, b1.reshape(1, D), W2l, W2r)

    num2, seg2 = _edge_layer(src, dst, xl2, xr2, a2, zeros_rows, zeros_flat)
    s2a = seg2[:NP, None]
    s2b = seg2[NP:, None]

    out = pl.pallas_call(
        _final_body,
        out_shape=jax.ShapeDtypeStruct((NP, D), _f32),
    )(num2, s2a, s2b, b2.reshape(1, D))
    return out[:N]


# speedup vs baseline: 8.1965x; 1.0317x over previous
import functools

import jax
import jax.numpy as jnp
from jax import lax
from jax.experimental import pallas as pl
from jax.experimental.pallas import tpu as pltpu
from jax.experimental.pallas import tpu_sc as plsc

N = 10000
E = 320000
D_IN = 128
D = 32

NC = 2
NS = 16
NW = NC * NS
NP = 10240
EPW = E // NW
C = 80
NCHUNK = EPW // C
NDBL = (NCHUNK - 1) // 2
G = C // 16
RPS = NP // NS

_f32 = jnp.float32
_i32 = jnp.int32

_MESH = plsc.VectorSubcoreMesh(core_axis_name="c", subcore_axis_name="s")
_SC_PARAMS = pltpu.CompilerParams(
    needs_layout_passes=False, use_tc_tiling_on_sc=False)


def _edge_body(src_hbm, dst_hbm, xl_hbm, xr_hbm, att_hbm, zrows_hbm,
               zflat_hbm,
               num_hbm, seg_hbm,
               num_sh, seg_sh, xl_sp, xr_sp, att_v, idxs_all, idxd_all,
               isc0, idc0, isc1, idc1, sdc0, sdc1,
               rows_s0, rows_d0, rows_s1, rows_d1,
               wrows0, wrows1, alc0, alc1,
               sem0, sem1, semw0, semw1):
    c = lax.axis_index("c")
    s = lax.axis_index("s")
    ebase = (s * NC + c) * EPW

    pltpu.sync_copy(zrows_hbm, num_sh.at[pl.ds(s * RPS, RPS)])
    pltpu.sync_copy(zflat_hbm, seg_sh.at[pl.ds(s * RPS, RPS)])
    pltpu.sync_copy(xl_hbm.at[pl.ds(s * RPS, RPS)], xl_sp.at[pl.ds(s * RPS, RPS)])
    pltpu.sync_copy(xr_hbm.at[pl.ds(s * RPS, RPS)], xr_sp.at[pl.ds(s * RPS, RPS)])
    pltpu.sync_copy(att_hbm, att_v)
    pltpu.sync_copy(src_hbm.at[pl.ds(ebase, EPW)], idxs_all)
    pltpu.sync_copy(dst_hbm.at[pl.ds(ebase, EPW)], idxd_all)
    plsc.subcore_barrier()

    a_lo = att_v[pl.ds(0, 16)]
    a_hi = att_v[pl.ds(16, 16)]
    atts = [a_lo[f] for f in range(16)] + [a_hi[f] for f in range(16)]

    bufs = [(isc0, idc0, sdc0, rows_s0, rows_d0, wrows0, alc0, sem0, semw0),
            (isc1, idc1, sdc1, rows_s1, rows_d1, wrows1, alc1, sem1, semw1)]

    def issue(i, p):
        isc, idc, _, rs, rd, _, _, sm, _ = bufs[p]
        for g in range(G):
            isc[pl.ds(g * 16, 16)] = idxs_all[pl.ds(i * C + g * 16, 16)]
            idc[pl.ds(g * 16, 16)] = idxd_all[pl.ds(i * C + g * 16, 16)]
        pltpu.async_copy(xl_sp.at[isc], rs, sm)
        pltpu.async_copy(xr_sp.at[idc], rd, sm)

    def wait(p):
        isc, idc, _, rs, rd, _, _, sm, _ = bufs[p]
        pltpu.make_async_copy(xl_sp.at[isc], rs, sm).wait()
        pltpu.make_async_copy(xr_sp.at[idc], rd, sm).wait()

    def wait_scatter(p):
        _, _, sdc, _, _, wrows, alc, _, smw = bufs[p]
        pltpu.make_async_copy(alc, seg_sh.at[sdc], smw).wait()
        pltpu.make_async_copy(wrows, num_sh.at[sdc], smw).wait()

    def process(i, p):
        isc, idc, sdc, rs, rd, wrows, alc, sm, smw = bufs[p]
        for g in range(G):
            eids = lax.iota(_i32, 16) + _i32(g * 16)
            sdc[pl.ds(g * 16, 16)] = idc[pl.ds(g * 16, 16)]
            accs = [jnp.zeros((16,), _f32) for _ in range(4)]
            for f in range(D):
                fv = jnp.full((16,), f, _i32)
                sv = (plsc.load_gather(rs, [eids, fv])
                      + plsc.load_gather(rd, [eids, fv]))
                m = jnp.maximum(sv, 0.2 * sv)
                accs[f % 4] = accs[f % 4] + atts[f] * m
            alpha = jnp.exp((accs[0] + accs[1]) + (accs[2] + accs[3]))
            alc[pl.ds(g * 16, 16)] = alpha
            for f in range(D):
                fv = jnp.full((16,), f, _i32)
                v = plsc.load_gather(rs, [eids, fv]) * alpha
                plsc.store_scatter(wrows, [eids, fv], v)
        pltpu.async_copy(alc, seg_sh.at[sdc], smw, add=True)
        pltpu.async_copy(wrows, num_sh.at[sdc], smw, add=True)

    issue(0, 0)

    def body(j, carry):
        i0 = 2 * j
        wait(0)
        issue(i0 + 1, 1)

        @pl.when(j != 0)
        def _():
            wait_scatter(0)

        process(i0, 0)
        wait(1)
        issue(i0 + 2, 0)

        @pl.when(j != 0)
        def _():
            wait_scatter(1)

        process(i0 + 1, 1)
        return carry

    lax.fori_loop(0, NDBL, body, 0)
    wait(0)
    wait_scatter(0)
    process(NCHUNK - 1, 0)
    wait_scatter(1)
    wait_scatter(0)

    plsc.subcore_barrier()
    pltpu.sync_copy(num_sh.at[pl.ds(s * RPS, RPS)],
                    num_hbm.at[pl.ds(c * NP + s * RPS, RPS)])
    pltpu.sync_copy(seg_sh.at[pl.ds(s * RPS, RPS)],
                    seg_hbm.at[pl.ds(c * NP + s * RPS, RPS)])


_edge_layer = functools.partial(
    pl.kernel,
    out_type=[jax.ShapeDtypeStruct((NC * NP, D), _f32),
              jax.ShapeDtypeStruct((NC * NP,), _f32)],
    mesh=_MESH,
    scratch_types=[
        pltpu.VMEM_SHARED((NP, D), _f32),
        pltpu.VMEM_SHARED((NP,), _f32),
        pltpu.VMEM_SHARED((NP, D), _f32),
        pltpu.VMEM_SHARED((NP, D), _f32),
        pltpu.VMEM((D,), _f32),
        pltpu.VMEM((EPW,), _i32),
        pltpu.VMEM((EPW,), _i32),
        pltpu.VMEM((C,), _i32),
        pltpu.VMEM((C,), _i32),
        pltpu.VMEM((C,), _i32),
        pltpu.VMEM((C,), _i32),
        pltpu.VMEM((C,), _i32),
        pltpu.VMEM((C,), _i32),
        pltpu.VMEM((C, D), _f32),
        pltpu.VMEM((C, D), _f32),
        pltpu.VMEM((C, D), _f32),
        pltpu.VMEM((C, D), _f32),
        pltpu.VMEM((C, D), _f32),
        pltpu.VMEM((C, D), _f32),
        pltpu.VMEM((C,), _f32),
        pltpu.VMEM((C,), _f32),
        pltpu.SemaphoreType.DMA,
        pltpu.SemaphoreType.DMA,
        pltpu.SemaphoreType.DMA,
        pltpu.SemaphoreType.DMA,
    ],
    compiler_params=_SC_PARAMS,
)(_edge_body)


def _mm2_body(x_ref, wl_ref, wr_ref, xl_ref, xr_ref):
    x = x_ref[...]
    xl_ref[...] = jnp.dot(x, wl_ref[...], preferred_element_type=_f32)
    xr_ref[...] = jnp.dot(x, wr_ref[...], preferred_element_type=_f32)


def _layer2_body(p_ref, s0_ref, s1_ref, b_ref, wl_ref, wr_ref,
                 xl_ref, xr_ref):
    p = p_ref[...]
    seg = s0_ref[...] + s1_ref[...] + 1e-16
    h = jnp.maximum((p[:NP] + p[NP:]) / seg + b_ref[...], 0.0)
    xl_ref[...] = jnp.dot(h, wl_ref[...], preferred_element_type=_f32)
    xr_ref[...] = jnp.dot(h, wr_ref[...], preferred_element_type=_f32)


def _final_body(p_ref, s0_ref, s1_ref, b_ref, o_ref):
    p = p_ref[...]
    seg = s0_ref[...] + s1_ref[...] + 1e-16
    o_ref[...] = (p[:NP] + p[NP:]) / seg + b_ref[...]


def kernel(x, edge_index, W1l, W1r, a1, b1, W2l, W2r, a2, b2):
    src = edge_index[0].astype(_i32)
    dst = edge_index[1].astype(_i32)
    xpad = jnp.pad(x.astype(_f32), ((0, NP - N), (0, 0)))
    zeros_rows = jnp.zeros((RPS, D), _f32)
    zeros_flat = jnp.zeros((RPS,), _f32)

    xl1, xr1 = pl.pallas_call(
        _mm2_body,
        out_shape=[jax.ShapeDtypeStruct((NP, D), _f32)] * 2,
    )(xpad, W1l, W1r)

    num1, seg1 = _edge_layer(src, dst, xl1, xr1, a1, zeros_rows, zeros_flat)
    s1a = seg1[:NP, None]
    s1b = seg1[NP:, None]

    xl2, xr2 = pl.pallas_call(
        _layer2_body,
        out_shape=[jax.ShapeDtypeStruct((NP, D), _f32)] * 2,
    )(num1, s1a, s1b, b1.reshape(1, D), W2l, W2r)

    num2, seg2 = _edge_layer(src, dst, xl2, xr2, a2, zeros_rows, zeros_flat)
    s2a = seg2[:NP, None]
    s2b = seg2[NP:, None]

    out = pl.pallas_call(
        _final_body,
        out_shape=jax.ShapeDtypeStruct((NP, D), _f32),
    )(num2, s2a, s2b, b2.reshape(1, D))
    return out[:N]
